# Initial kernel scaffold; baseline (speedup 1.0000x reference)
import jax
import jax.numpy as jnp
from jax.experimental import pallas as pl


def kernel(words, mask, embed_table, lin_w, lin_b):
    raise NotImplementedError("write your pallas kernel here")


# speedup vs baseline: 14.6337x; 14.6337x over previous
import functools

import jax
import jax.numpy as jnp
from jax import lax
from jax.experimental import pallas as pl
from jax.experimental.pallas import tpu as pltpu
from jax.experimental.pallas import tpu_sc as plsc

VOCAB = 100000
EMB = 64
BATCH = 16384
SEQ = 200
NCLS = 20
PD = 32
NC, NS = 2, 16
NW = NC * NS
ROWS_PER_W = BATCH // NW
G = 8
HALF = SEQ // 2


def _proj_kernel(t_ref, w_ref, o_ref):
    o_ref[...] = jnp.dot(t_ref[...], w_ref[...],
                         preferred_element_type=jnp.float32)


def _project_table(table, wpad):
    blk = 1000
    return pl.pallas_call(
        _proj_kernel,
        grid=(VOCAB // blk,),
        in_specs=[
            pl.BlockSpec((blk, EMB), lambda i: (i, 0)),
            pl.BlockSpec((EMB, PD), lambda i: (0, 0)),
        ],
        out_specs=pl.BlockSpec((blk, PD), lambda i: (i, 0)),
        out_shape=jax.ShapeDtypeStruct((VOCAB, PD), jnp.float32),
    )(table, wpad)


def _sc_pool(ptable, words3):
    mesh = plsc.VectorSubcoreMesh(core_axis_name="c", subcore_axis_name="s",
                                  num_cores=NC, num_subcores=NS)

    @functools.partial(
        pl.kernel,
        out_type=jax.ShapeDtypeStruct((BATCH, PD), jnp.float32),
        mesh=mesh,
        compiler_params=pltpu.CompilerParams(use_tc_tiling_on_sc=False),
        scratch_types=[
            pltpu.VMEM((G, 2, HALF), jnp.int32),
            pltpu.VMEM((2, HALF, PD), jnp.float32),
            pltpu.VMEM((ROWS_PER_W, PD), jnp.float32),
            pltpu.SemaphoreType.DMA,
        ],
    )
    def sc_k(pt_hbm, w_hbm, out_hbm, idx_v, rows_v, out_v, sem):
        wid = lax.axis_index("s") * NC + lax.axis_index("c")
        base = wid * ROWS_PER_W

        def chunk(g, carry):
            rowbase = base + g * G
            pltpu.sync_copy(w_hbm.at[pl.ds(rowbase, G)], idx_v)
            for r in range(G):
                h0 = pltpu.async_copy(pt_hbm.at[idx_v.at[r, 0]],
                                      rows_v.at[0], sem)
                h1 = pltpu.async_copy(pt_hbm.at[idx_v.at[r, 1]],
                                      rows_v.at[1], sem)
                h0.wait()
                h1.wait()

                z = jnp.zeros((16,), jnp.float32)

                def red(s, accs):
                    a0, a1 = accs
                    a0 = a0 + rows_v[0, s, pl.ds(0, 16)]
                    a1 = a1 + rows_v[0, s, pl.ds(16, 16)]
                    a0 = a0 + rows_v[1, s, pl.ds(0, 16)]
                    a1 = a1 + rows_v[1, s, pl.ds(16, 16)]
                    return (a0, a1)

                a0, a1 = lax.fori_loop(0, HALF, red, (z, z))
                orow = g * G + r
                out_v[orow, pl.ds(0, 16)] = a0
                out_v[orow, pl.ds(16, 16)] = a1
            return carry

        lax.fori_loop(0, ROWS_PER_W // G, chunk, 0)
        pltpu.sync_copy(out_v, out_hbm.at[pl.ds(base, ROWS_PER_W)])

    return sc_k(ptable, words3)


def _finish_kernel(s_ref, m_ref, b_ref, o_ref):
    denom = jnp.sum(m_ref[...], axis=1, keepdims=True)
    o_ref[...] = s_ref[...][:, :NCLS] / denom + b_ref[...]


def _finish(sums, mask, bias2d):
    blk = 2048
    return pl.pallas_call(
        _finish_kernel,
        grid=(BATCH // blk,),
        in_specs=[
            pl.BlockSpec((blk, PD), lambda i: (i, 0)),
            pl.BlockSpec((blk, SEQ), lambda i: (i, 0)),
            pl.BlockSpec((1, NCLS), lambda i: (0, 0)),
        ],
        out_specs=pl.BlockSpec((blk, NCLS), lambda i: (i, 0)),
        out_shape=jax.ShapeDtypeStruct((BATCH, NCLS), jnp.float32),
    )(sums, mask, bias2d)


def kernel(words, mask, embed_table, lin_w, lin_b):
    wpad = jnp.zeros((EMB, PD), jnp.float32).at[:, :NCLS].set(lin_w.T)
    ptable = _project_table(embed_table, wpad)
    words3 = words.astype(jnp.int32).reshape(BATCH, 2, HALF)
    sums = _sc_pool(ptable, words3)
    return _finish(sums, mask, lin_b.reshape(1, NCLS))


# speedup vs baseline: 29.2551x; 1.9992x over previous
import functools

import jax
import jax.numpy as jnp
from jax import lax
from jax.experimental import pallas as pl
from jax.experimental.pallas import tpu as pltpu
from jax.experimental.pallas import tpu_sc as plsc

VOCAB = 100000
EMB = 64
BATCH = 16384
SEQ = 200
NCLS = 20
PD = 32
NC, NS = 2, 16
NW = NC * NS
ROWS_PER_W = BATCH // NW
G = 8
HALF = SEQ // 2


def _proj_kernel(t_ref, w_ref, o_ref):
    o_ref[...] = jnp.dot(t_ref[...], w_ref[...],
                         preferred_element_type=jnp.float32)


def _project_table(table, wpad):
    blk = 1000
    return pl.pallas_call(
        _proj_kernel,
        grid=(VOCAB // blk,),
        in_specs=[
            pl.BlockSpec((blk, EMB), lambda i: (i, 0)),
            pl.BlockSpec((EMB, PD), lambda i: (0, 0)),
        ],
        out_specs=pl.BlockSpec((blk, PD), lambda i: (i, 0)),
        out_shape=jax.ShapeDtypeStruct((VOCAB, PD), jnp.float32),
    )(table, wpad)


def _sc_pool(ptable, words3):
    mesh = plsc.VectorSubcoreMesh(core_axis_name="c", subcore_axis_name="s",
                                  num_cores=NC, num_subcores=NS)
    CH = ROWS_PER_W // (2 * G)
    U = 4

    @functools.partial(
        pl.kernel,
        out_type=jax.ShapeDtypeStruct((BATCH, PD), jnp.float32),
        mesh=mesh,
        compiler_params=pltpu.CompilerParams(use_tc_tiling_on_sc=False),
        scratch_types=[
            pltpu.VMEM((2, G, 2, HALF), jnp.int32),
            pltpu.VMEM((2, G, 2, HALF, PD), jnp.float32),
            pltpu.VMEM((ROWS_PER_W, PD), jnp.float32),
            pltpu.SemaphoreType.DMA,
            pltpu.SemaphoreType.DMA,
        ],
    )
    def sc_k(pt_hbm, w_hbm, out_hbm, idx_v, rows_v, out_v, sem0, sem1):
        wid = lax.axis_index("s") * NC + lax.axis_index("c")
        base = wid * ROWS_PER_W
        sems = (sem0, sem1)

        def load_idx(chunk, buf):
            pltpu.sync_copy(w_hbm.at[pl.ds(base + chunk * G, G)],
                            idx_v.at[buf])

        def issue(buf):
            for r in range(G):
                for h in range(2):
                    pltpu.async_copy(pt_hbm.at[idx_v.at[buf, r, h]],
                                     rows_v.at[buf, r, h], sems[buf])

        def drain(buf):
            for r in range(G):
                for h in range(2):
                    pltpu.make_async_copy(pt_hbm.at[idx_v.at[buf, r, h]],
                                          rows_v.at[buf, r, h],
                                          sems[buf]).wait()

        def reduce_chunk(buf, chunk):
            for r in range(G):
                z = jnp.zeros((16,), jnp.float32)

                def red(i, accs):
                    a00, a01, a10, a11 = accs
                    for u in range(U):
                        s = i * U + u
                        a00 = a00 + rows_v[buf, r, 0, s, pl.ds(0, 16)]
                        a01 = a01 + rows_v[buf, r, 0, s, pl.ds(16, 16)]
                        a10 = a10 + rows_v[buf, r, 1, s, pl.ds(0, 16)]
                        a11 = a11 + rows_v[buf, r, 1, s, pl.ds(16, 16)]
                    return (a00, a01, a10, a11)

                a00, a01, a10, a11 = lax.fori_loop(0, HALF // U, red,
                                                   (z, z, z, z))
                orow = chunk * G + r
                out_v[orow, pl.ds(0, 16)] = a00 + a10
                out_v[orow, pl.ds(16, 16)] = a01 + a11

        load_idx(0, 0)
        issue(0)
        load_idx(1, 1)
        issue(1)

        def body(k, carry):
            drain(0)
            reduce_chunk(0, 2 * k)

            @pl.when(k < CH - 1)
            def _():
                load_idx(2 * k + 2, 0)
                issue(0)

            drain(1)
            reduce_chunk(1, 2 * k + 1)

            @pl.when(k < CH - 1)
            def _():
                load_idx(2 * k + 3, 1)
                issue(1)

            return carry

        lax.fori_loop(0, CH, body, 0)
        pltpu.sync_copy(out_v, out_hbm.at[pl.ds(base, ROWS_PER_W)])

    return sc_k(ptable, words3)


def _finish_kernel(s_ref, m_ref, b_ref, o_ref):
    denom = jnp.sum(m_ref[...], axis=1, keepdims=True)
    o_ref[...] = s_ref[...][:, :NCLS] / denom + b_ref[...]


def _finish(sums, mask, bias2d):
    blk = 2048
    return pl.pallas_call(
        _finish_kernel,
        grid=(BATCH // blk,),
        in_specs=[
            pl.BlockSpec((blk, PD), lambda i: (i, 0)),
            pl.BlockSpec((blk, SEQ), lambda i: (i, 0)),
            pl.BlockSpec((1, NCLS), lambda i: (0, 0)),
        ],
        out_specs=pl.BlockSpec((blk, NCLS), lambda i: (i, 0)),
        out_shape=jax.ShapeDtypeStruct((BATCH, NCLS), jnp.float32),
    )(sums, mask, bias2d)


def kernel(words, mask, embed_table, lin_w, lin_b):
    wpad = jnp.zeros((EMB, PD), jnp.float32).at[:, :NCLS].set(lin_w.T)
    ptable = _project_table(embed_table, wpad)
    words3 = words.astype(jnp.int32).reshape(BATCH, 2, HALF)
    sums = _sc_pool(ptable, words3)
    return _finish(sums, mask, lin_b.reshape(1, NCLS))


# speedup vs baseline: 34.4225x; 1.1766x over previous
import functools

import jax
import jax.numpy as jnp
from jax import lax
from jax.experimental import pallas as pl
from jax.experimental.pallas import tpu as pltpu
from jax.experimental.pallas import tpu_sc as plsc

VOCAB = 100000
EMB = 64
BATCH = 16384
SEQ = 200
NCLS = 20
PD = 32
NC, NS = 2, 16
NW = NC * NS
ROWS_PER_W = BATCH // NW
G = 8
H0, H1 = 104, 96


def _proj_kernel(t_ref, w_ref, o_ref):
    p = lax.dot_general(t_ref[...], w_ref[...], (((0,), (0,)), ((), ())),
                        preferred_element_type=jnp.float32)
    o_ref[...] = p.astype(jnp.bfloat16)


def _project_table(table_t, wpad):
    blk = 12544
    return pl.pallas_call(
        _proj_kernel,
        grid=(8,),
        in_specs=[
            pl.BlockSpec((EMB, blk), lambda i: (0, i)),
            pl.BlockSpec((EMB, PD), lambda i: (0, 0)),
        ],
        out_specs=pl.BlockSpec((blk, PD), lambda i: (i, 0)),
        out_shape=jax.ShapeDtypeStruct((VOCAB, PD), jnp.bfloat16),
    )(table_t, wpad)


def _sc_pool(ptable_w, words):
    mesh = plsc.VectorSubcoreMesh(core_axis_name="c", subcore_axis_name="s",
                                  num_cores=NC, num_subcores=NS)
    CH = ROWS_PER_W // (2 * G)
    U = 8
    PW = PD // 2

    @functools.partial(
        pl.kernel,
        out_type=jax.ShapeDtypeStruct((BATCH, PD), jnp.float32),
        mesh=mesh,
        compiler_params=pltpu.CompilerParams(use_tc_tiling_on_sc=False,
                                             needs_layout_passes=False),
        scratch_types=[
            pltpu.VMEM((2, G, SEQ), jnp.int32),
            pltpu.VMEM((2, G, 2, H0, PW), jnp.int32),
            pltpu.VMEM((ROWS_PER_W, PD), jnp.float32),
            pltpu.SemaphoreType.DMA,
            pltpu.SemaphoreType.DMA,
        ],
    )
    def sc_k(pt_hbm, w_hbm, out_hbm, idx_v, rows_v, out_v, sem0, sem1):
        wid = lax.axis_index("s") * NC + lax.axis_index("c")
        base = wid * ROWS_PER_W
        sems = (sem0, sem1)

        def load_idx(chunk, buf):
            pltpu.sync_copy(w_hbm.at[pl.ds(base + chunk * G, G)],
                            idx_v.at[buf])

        def copies(buf):
            for r in range(G):
                yield (pt_hbm.at[idx_v.at[buf, r, pl.ds(0, H0)]],
                       rows_v.at[buf, r, 0], sems[buf])
                yield (pt_hbm.at[idx_v.at[buf, r, pl.ds(H0, H1)]],
                       rows_v.at[buf, r, 1, pl.ds(0, H1)], sems[buf])

        def issue(buf):
            for src, dst, sem in copies(buf):
                pltpu.async_copy(src, dst, sem)

        def drain(buf):
            for src, dst, sem in copies(buf):
                pltpu.make_async_copy(src, dst, sem).wait()

        def lo(x):
            return plsc.bitcast(x << 16, jnp.float32)

        def hi(x):
            return plsc.bitcast(x, jnp.float32)

        def reduce_chunk(buf, chunk):
            for r in range(G):
                z = jnp.zeros((16,), jnp.float32)

                def red(i, accs):
                    al0, ah0, al1, ah1 = accs
                    for u in range(U):
                        s = i * U + u
                        x0 = rows_v[buf, r, 0, s, :]
                        x1 = rows_v[buf, r, 1, s, :]
                        al0 = al0 + lo(x0)
                        ah0 = ah0 + hi(x0)
                        al1 = al1 + lo(x1)
                        ah1 = ah1 + hi(x1)
                    return (al0, ah0, al1, ah1)

                al0, ah0, al1, ah1 = lax.fori_loop(0, H1 // U, red,
                                                   (z, z, z, z))
                for s in range(H1, H0):
                    x0 = rows_v[buf, r, 0, s, :]
                    al0 = al0 + lo(x0)
                    ah0 = ah0 + hi(x0)
                orow = chunk * G + r
                out_v[orow, pl.ds(0, PW)] = al0 + al1
                out_v[orow, pl.ds(PW, PW)] = ah0 + ah1

        load_idx(0, 0)
        issue(0)
        load_idx(1, 1)
        issue(1)

        def body(k, carry):
            drain(0)
            reduce_chunk(0, 2 * k)

            @pl.when(k < CH - 1)
            def _():
                load_idx(2 * k + 2, 0)
                issue(0)

            drain(1)
            reduce_chunk(1, 2 * k + 1)

            @pl.when(k < CH - 1)
            def _():
                load_idx(2 * k + 3, 1)
                issue(1)

            return carry

        lax.fori_loop(0, CH, body, 0)
        pltpu.sync_copy(out_v, out_hbm.at[pl.ds(base, ROWS_PER_W)])

    return sc_k(ptable_w, words)


def _finish_kernel(s_ref, mt_ref, b_ref, o_ref):
    denom = jnp.sum(mt_ref[...], axis=0)
    o_ref[...] = s_ref[...][:, :NCLS] / denom[:, None] + b_ref[...]


def _finish(sums, mask_t, bias2d):
    blk = 2048
    return pl.pallas_call(
        _finish_kernel,
        grid=(BATCH // blk,),
        in_specs=[
            pl.BlockSpec((blk, PD), lambda i: (i, 0)),
            pl.BlockSpec((SEQ, blk), lambda i: (0, i)),
            pl.BlockSpec((1, NCLS), lambda i: (0, 0)),
        ],
        out_specs=pl.BlockSpec((blk, NCLS), lambda i: (i, 0)),
        out_shape=jax.ShapeDtypeStruct((BATCH, NCLS), jnp.float32),
    )(sums, mask_t, bias2d)


def kernel(words, mask, embed_table, lin_w, lin_b):
    cols = jnp.array([2 * f for f in range(16)]
                     + [2 * (f - 16) + 1 for f in range(16, NCLS)])
    wpad = jnp.zeros((EMB, PD), jnp.float32).at[:, cols].set(lin_w.T)
    ptable = _project_table(embed_table.T, wpad)
    ptable_w = lax.bitcast_convert_type(
        ptable.reshape(VOCAB, PD // 2, 2), jnp.int32)
    sums = _sc_pool(ptable_w, words.astype(jnp.int32))
    return _finish(sums, mask.T, lin_b.reshape(1, NCLS))


# speedup vs baseline: 47.6527x; 1.3843x over previous
import functools

import jax
import jax.numpy as jnp
from jax import lax
from jax.experimental import pallas as pl
from jax.experimental.pallas import tpu as pltpu
from jax.experimental.pallas import tpu_sc as plsc

VOCAB = 100000
EMB = 64
BATCH = 16384
SEQ = 200
NCLS = 20
PD = 32
NC, NS = 2, 16
NW = NC * NS
ROWS_PER_W = BATCH // NW
G = 8
H0, H1 = 104, 96


def _proj_kernel(t_ref, w_ref, o_ref):
    p = lax.dot_general(t_ref[...], w_ref[...], (((0,), (0,)), ((), ())),
                        preferred_element_type=jnp.float32)
    o_ref[...] = p.astype(jnp.bfloat16)


def _project_table(table_t, wpad):
    blk = 12544
    return pl.pallas_call(
        _proj_kernel,
        grid=(8,),
        in_specs=[
            pl.BlockSpec((EMB, blk), lambda i: (0, i)),
            pl.BlockSpec((EMB, PD), lambda i: (0, 0)),
        ],
        out_specs=pl.BlockSpec((blk, PD), lambda i: (i, 0)),
        out_shape=jax.ShapeDtypeStruct((VOCAB, PD), jnp.bfloat16),
    )(table_t, wpad)


def _sc_pool(ptable_w, words):
    mesh = plsc.VectorSubcoreMesh(core_axis_name="c", subcore_axis_name="s",
                                  num_cores=NC, num_subcores=NS)
    CH = ROWS_PER_W // (2 * G)
    U = 8
    PW = PD // 2

    @functools.partial(
        pl.kernel,
        out_type=jax.ShapeDtypeStruct((BATCH, PD), jnp.float32),
        mesh=mesh,
        compiler_params=pltpu.CompilerParams(use_tc_tiling_on_sc=False,
                                             needs_layout_passes=False),
        scratch_types=[
            pltpu.VMEM((2, G, SEQ), jnp.int32),
            pltpu.VMEM((2, G, 2, H0, PD), jnp.bfloat16),
            pltpu.VMEM((ROWS_PER_W, PD), jnp.float32),
            pltpu.SemaphoreType.DMA,
            pltpu.SemaphoreType.DMA,
        ],
    )
    def sc_k(pt_hbm, w_hbm, out_hbm, idx_v, rows_v, out_v, sem0, sem1):
        wid = lax.axis_index("s") * NC + lax.axis_index("c")
        base = wid * ROWS_PER_W
        sems = (sem0, sem1)

        def load_idx(chunk, buf):
            pltpu.sync_copy(w_hbm.at[pl.ds(base + chunk * G, G)],
                            idx_v.at[buf])

        def copies(buf):
            for r in range(G):
                yield (pt_hbm.at[idx_v.at[buf, r, pl.ds(0, H0)]],
                       rows_v.at[buf, r, 0], sems[buf])
                yield (pt_hbm.at[idx_v.at[buf, r, pl.ds(H0, H1)]],
                       rows_v.at[buf, r, 1, pl.ds(0, H1)], sems[buf])

        def issue(buf):
            for src, dst, sem in copies(buf):
                pltpu.async_copy(src, dst, sem)

        def drain(buf):
            for src, dst, sem in copies(buf):
                pltpu.make_async_copy(src, dst, sem).wait()

        def lo(x):
            return plsc.bitcast(x << 16, jnp.float32)

        def hi(x):
            return plsc.bitcast(x, jnp.float32)

        def reduce_chunk(buf, chunk):
            for r in range(G):
                z = jnp.zeros((16,), jnp.float32)

                def red(i, accs):
                    al0, ah0, al1, ah1 = accs
                    for u in range(U):
                        s = i * U + u
                        x0 = plsc.bitcast(rows_v[buf, r, 0, s, :], jnp.int32)
                        x1 = plsc.bitcast(rows_v[buf, r, 1, s, :], jnp.int32)
                        al0 = al0 + lo(x0)
                        ah0 = ah0 + hi(x0)
                        al1 = al1 + lo(x1)
                        ah1 = ah1 + hi(x1)
                    return (al0, ah0, al1, ah1)

                al0, ah0, al1, ah1 = lax.fori_loop(0, H1 // U, red,
                                                   (z, z, z, z))
                for s in range(H1, H0):
                    x0 = plsc.bitcast(rows_v[buf, r, 0, s, :], jnp.int32)
                    al0 = al0 + lo(x0)
                    ah0 = ah0 + hi(x0)
                orow = chunk * G + r
                out_v[orow, pl.ds(0, PW)] = al0 + al1
                out_v[orow, pl.ds(PW, PW)] = ah0 + ah1

        load_idx(0, 0)
        issue(0)
        load_idx(1, 1)
        issue(1)

        def body(k, carry):
            drain(0)
            reduce_chunk(0, 2 * k)

            @pl.when(k < CH - 1)
            def _():
                load_idx(2 * k + 2, 0)
                issue(0)

            drain(1)
            reduce_chunk(1, 2 * k + 1)

            @pl.when(k < CH - 1)
            def _():
                load_idx(2 * k + 3, 1)
                issue(1)

            return carry

        lax.fori_loop(0, CH, body, 0)
        pltpu.sync_copy(out_v, out_hbm.at[pl.ds(base, ROWS_PER_W)])

    return sc_k(ptable_w, words)


def _finish_kernel(s_ref, mt_ref, b_ref, o_ref):
    denom = jnp.sum(mt_ref[...], axis=0)
    o_ref[...] = s_ref[...][:, :NCLS] / denom[:, None] + b_ref[...]


def _finish(sums, mask_t, bias2d):
    blk = 2048
    return pl.pallas_call(
        _finish_kernel,
        grid=(BATCH // blk,),
        in_specs=[
            pl.BlockSpec((blk, PD), lambda i: (i, 0)),
            pl.BlockSpec((SEQ, blk), lambda i: (0, i)),
            pl.BlockSpec((1, NCLS), lambda i: (0, 0)),
        ],
        out_specs=pl.BlockSpec((blk, NCLS), lambda i: (i, 0)),
        out_shape=jax.ShapeDtypeStruct((BATCH, NCLS), jnp.float32),
    )(sums, mask_t, bias2d)


def kernel(words, mask, embed_table, lin_w, lin_b):
    cols = jnp.array([2 * f for f in range(16)]
                     + [2 * (f - 16) + 1 for f in range(16, NCLS)])
    wpad = jnp.zeros((EMB, PD), jnp.float32).at[:, cols].set(lin_w.T)
    ptable = _project_table(embed_table.T, wpad)
    sums = _sc_pool(ptable, words.astype(jnp.int32))
    return _finish(sums, mask.T, lin_b.reshape(1, NCLS))


# speedup vs baseline: 49.5838x; 1.0405x over previous
import functools

import jax
import jax.numpy as jnp
from jax import lax
from jax.experimental import pallas as pl
from jax.experimental.pallas import tpu as pltpu
from jax.experimental.pallas import tpu_sc as plsc

VOCAB = 100000
EMB = 64
BATCH = 16384
SEQ = 200
NCLS = 20
PD = 32
NC, NS = 2, 16
NW = NC * NS
ROWS_PER_W = BATCH // NW
G = 16
FL = G * SEQ
NSTR = FL // 128


def _proj_kernel(t_ref, w_ref, o_ref):
    p = lax.dot_general(t_ref[...], w_ref[...], (((0,), (0,)), ((), ())),
                        preferred_element_type=jnp.float32)
    o_ref[...] = p.astype(jnp.bfloat16)


def _project_table(table_t, wpad):
    blk = 12544
    return pl.pallas_call(
        _proj_kernel,
        grid=(8,),
        in_specs=[
            pl.BlockSpec((EMB, blk), lambda i: (0, i)),
            pl.BlockSpec((EMB, PD), lambda i: (0, 0)),
        ],
        out_specs=pl.BlockSpec((blk, PD), lambda i: (i, 0)),
        out_shape=jax.ShapeDtypeStruct((VOCAB, PD), jnp.bfloat16),
    )(table_t, wpad)


def _sc_pool(ptable, words1d):
    mesh = plsc.VectorSubcoreMesh(core_axis_name="c", subcore_axis_name="s",
                                  num_cores=NC, num_subcores=NS)
    CH = ROWS_PER_W // (2 * G)
    U = 10

    @functools.partial(
        pl.kernel,
        out_type=jax.ShapeDtypeStruct((BATCH, PD), jnp.float32),
        mesh=mesh,
        compiler_params=pltpu.CompilerParams(use_tc_tiling_on_sc=False,
                                             needs_layout_passes=False),
        scratch_types=[
            pltpu.VMEM((2, FL), jnp.int32),
            pltpu.VMEM((2, FL, PD), jnp.bfloat16),
            pltpu.VMEM((ROWS_PER_W, PD), jnp.float32),
            pltpu.SemaphoreType.DMA,
            pltpu.SemaphoreType.DMA,
        ],
    )
    def sc_k(pt_hbm, w_hbm, out_hbm, idx_v, rows_v, out_v, sem0, sem1):
        wid = lax.axis_index("s") * NC + lax.axis_index("c")
        base = wid * ROWS_PER_W
        sems = (sem0, sem1)

        def load_idx(chunk, buf):
            pltpu.sync_copy(w_hbm.at[pl.ds((base + chunk * G) * SEQ, FL)],
                            idx_v.at[buf])

        def copies(buf):
            for j in range(NSTR):
                yield (pt_hbm.at[idx_v.at[buf, pl.ds(128 * j, 128)]],
                       rows_v.at[buf, pl.ds(128 * j, 128)], sems[buf])

        def issue(buf):
            for src, dst, sem in copies(buf):
                pltpu.async_copy(src, dst, sem)

        def drain(buf):
            for src, dst, sem in copies(buf):
                pltpu.make_async_copy(src, dst, sem).wait()

        def lo(x):
            return plsc.bitcast(x << 16, jnp.float32)

        def hi(x):
            return plsc.bitcast(x, jnp.float32)

        def reduce_chunk(buf, chunk):
            for r in range(G):
                z = jnp.zeros((16,), jnp.float32)

                def red(i, accs):
                    al0, ah0, al1, ah1 = accs
                    for u in range(U):
                        s = r * SEQ + i * (2 * U) + 2 * u
                        t = rows_v[buf, s, :] + rows_v[buf, s + 1, :]
                        w = plsc.bitcast(t, jnp.int32)
                        if u % 2 == 0:
                            al0 = al0 + lo(w)
                            ah0 = ah0 + hi(w)
                        else:
                            al1 = al1 + lo(w)
                            ah1 = ah1 + hi(w)
                    return (al0, ah0, al1, ah1)

                al0, ah0, al1, ah1 = lax.fori_loop(0, SEQ // (2 * U), red,
                                                   (z, z, z, z))
                orow = chunk * G + r
                out_v[orow, pl.ds(0, 16)] = al0 + al1
                out_v[orow, pl.ds(16, 16)] = ah0 + ah1

        load_idx(0, 0)
        issue(0)
        load_idx(1, 1)
        issue(1)

        def body(k, carry):
            drain(0)
            reduce_chunk(0, 2 * k)

            @pl.when(k < CH - 1)
            def _():
                load_idx(2 * k + 2, 0)
                issue(0)

            drain(1)
            reduce_chunk(1, 2 * k + 1)

            @pl.when(k < CH - 1)
            def _():
                load_idx(2 * k + 3, 1)
                issue(1)

            return carry

        lax.fori_loop(0, CH, body, 0)
        pltpu.sync_copy(out_v, out_hbm.at[pl.ds(base, ROWS_PER_W)])

    return sc_k(ptable, words1d)


def _finish_kernel(s_ref, mt_ref, b_ref, o_ref):
    denom = jnp.sum(mt_ref[...], axis=0)
    o_ref[...] = s_ref[...][:, :NCLS] / denom[:, None] + b_ref[...]


def _finish(sums, mask_t, bias2d):
    blk = 2048
    return pl.pallas_call(
        _finish_kernel,
        grid=(BATCH // blk,),
        in_specs=[
            pl.BlockSpec((blk, PD), lambda i: (i, 0)),
            pl.BlockSpec((SEQ, blk), lambda i: (0, i)),
            pl.BlockSpec((1, NCLS), lambda i: (0, 0)),
        ],
        out_specs=pl.BlockSpec((blk, NCLS), lambda i: (i, 0)),
        out_shape=jax.ShapeDtypeStruct((BATCH, NCLS), jnp.float32),
    )(sums, mask_t, bias2d)


def kernel(words, mask, embed_table, lin_w, lin_b):
    cols = jnp.array([2 * f for f in range(16)]
                     + [2 * (f - 16) + 1 for f in range(16, NCLS)])
    wpad = jnp.zeros((EMB, PD), jnp.float32).at[:, cols].set(lin_w.T)
    ptable = _project_table(embed_table.T, wpad)
    words1d = words.astype(jnp.int32).reshape(BATCH * SEQ)
    sums = _sc_pool(ptable, words1d)
    return _finish(sums, mask.T, lin_b.reshape(1, NCLS))
